# baseline (speedup 1.0000x reference)
import functools

import jax
import jax.numpy as jnp
from jax import lax
from jax.experimental import pallas as pl
from jax.experimental.pallas import tpu as pltpu
from jax.experimental.pallas import tpu_sc as plsc

_K = 8192
_D = 256
_BETA = 0.25
_M = 8192
_BM = 1024
_BN = 512
_N_TOK = _M // _BM
_N_CB = _K // _BN


def _argmin_body(z_ref, zsq_ref, cb_ref, csq_ref, idx_ref, rmin, ridx):
    j = pl.program_id(1)

    @pl.when(j == 0)
    def _():
        rmin[...] = jnp.full((_BM, 1), jnp.inf, jnp.float32)
        ridx[...] = jnp.zeros((_BM, 1), jnp.int32)

    mm = lax.dot_general(
        z_ref[...], cb_ref[...],
        dimension_numbers=(((1,), (1,)), ((), ())),
        preferred_element_type=jnp.float32,
    )
    dist = (zsq_ref[...] + csq_ref[...]) - 2.0 * mm
    lmin = jnp.min(dist, axis=1, keepdims=True)
    iota = lax.broadcasted_iota(jnp.int32, (_BM, _BN), 1)
    lidx = jnp.min(jnp.where(dist == lmin, iota, jnp.int32(2**30)),
                   axis=1, keepdims=True)
    gidx = lidx + j * _BN
    improved = lmin < rmin[...]
    rmin[...] = jnp.where(improved, lmin, rmin[...])
    ridx[...] = jnp.where(improved, gidx, ridx[...])

    @pl.when(j == _N_CB - 1)
    def _():
        idx_ref[...] = ridx[...]


def _argmin_indices(z_flat, z_sq, codebook, c_sq):
    return pl.pallas_call(
        _argmin_body,
        grid=(_N_TOK, _N_CB),
        in_specs=[
            pl.BlockSpec((_BM, _D), lambda i, j: (i, 0)),
            pl.BlockSpec((_BM, 1), lambda i, j: (i, 0)),
            pl.BlockSpec((_BN, _D), lambda i, j: (j, 0)),
            pl.BlockSpec((1, _BN), lambda i, j: (0, j)),
        ],
        out_specs=pl.BlockSpec((_BM, 1), lambda i, j: (i, 0)),
        out_shape=jax.ShapeDtypeStruct((_M, 1), jnp.int32),
        scratch_shapes=[
            pltpu.VMEM((_BM, 1), jnp.float32),
            pltpu.VMEM((_BM, 1), jnp.int32),
        ],
    )(z_flat, z_sq, codebook, c_sq)


_NC = 2
_NS = 16
_NW = _NC * _NS
_B_PER_W = _M // _NW


def _sc_gather_body(table_hbm, idx_hbm, out_hbm, idx_v, rows_v, sem):
    wid = lax.axis_index("s") * _NC + lax.axis_index("c")
    base = wid * _B_PER_W
    pltpu.sync_copy(idx_hbm.at[pl.ds(base, _B_PER_W)], idx_v)
    pltpu.async_copy(table_hbm.at[idx_v], rows_v, sem).wait()
    pltpu.sync_copy(rows_v, out_hbm.at[pl.ds(base, _B_PER_W)])


def _sc_gather(codebook, idx):
    kern = functools.partial(
        pl.kernel,
        mesh=plsc.VectorSubcoreMesh(core_axis_name="c", subcore_axis_name="s"),
        out_type=jax.ShapeDtypeStruct((_M, _D), jnp.float32),
        scratch_types=[
            pltpu.VMEM((_B_PER_W,), jnp.int32),
            pltpu.VMEM((_B_PER_W, _D), jnp.float32),
            pltpu.SemaphoreType.DMA,
        ],
    )(_sc_gather_body)
    return kern(codebook, idx)


_N_ELEMS = _M * _D


def _ste_body(z_ref, zq_ref, out_ref, loss_ref, acc):
    i = pl.program_id(0)

    @pl.when(i == 0)
    def _():
        acc[0, 0] = 0.0

    diff = zq_ref[...] - z_ref[...]
    out_ref[...] = z_ref[...] + diff
    acc[0, 0] += jnp.sum(diff * diff)

    @pl.when(i == _N_TOK - 1)
    def _():
        m = acc[0, 0] / _N_ELEMS
        loss_ref[...] = jnp.full((1, 1), _BETA * m + m, jnp.float32)


def _ste_and_loss(z_flat, zq_flat):
    return pl.pallas_call(
        _ste_body,
        grid=(_N_TOK,),
        in_specs=[
            pl.BlockSpec((_BM, _D), lambda i: (i, 0)),
            pl.BlockSpec((_BM, _D), lambda i: (i, 0)),
        ],
        out_specs=[
            pl.BlockSpec((_BM, _D), lambda i: (i, 0)),
            pl.BlockSpec((1, 1), lambda i: (0, 0)),
        ],
        out_shape=[
            jax.ShapeDtypeStruct((_M, _D), jnp.float32),
            jax.ShapeDtypeStruct((1, 1), jnp.float32),
        ],
        scratch_shapes=[pltpu.SMEM((1, 1), jnp.float32)],
    )(z_flat, zq_flat)


def kernel(z, codebook):
    z_ch = jnp.moveaxis(z, 1, -1)
    z_flat = z_ch.reshape(-1, _D)
    z_sq = jnp.sum(z_flat ** 2, axis=1, keepdims=True)
    c_sq = jnp.sum(codebook ** 2, axis=1).reshape(1, _K)

    idx2d = _argmin_indices(z_flat, z_sq, codebook, c_sq)
    zq_flat = _sc_gather(codebook, idx2d.reshape(_M))
    out_flat, loss = _ste_and_loss(z_flat, zq_flat)

    z_q_out = jnp.moveaxis(out_flat.reshape(z_ch.shape), -1, 1)
    return (z_q_out, loss.reshape(()))


# speedup vs baseline: 1.2518x; 1.2518x over previous
import functools

import jax
import jax.numpy as jnp
from jax import lax
from jax.experimental import pallas as pl
from jax.experimental.pallas import tpu as pltpu
from jax.experimental.pallas import tpu_sc as plsc

_K = 8192
_D = 256
_BETA = 0.25
_M = 8192
_BM = 2048
_BN = 2048
_N_TOK = _M // _BM
_N_CB = _K // _BN


def _argmin_body(zr_ref, zsq_ref, cb_ref, csq_ref, fiota_ref, idx_ref,
                 rmin, ridx):
    j = pl.program_id(1)

    @pl.when(j == 0)
    def _():
        rmin[...] = jnp.full((_BM, 1), jnp.inf, jnp.float32)
        ridx[...] = jnp.zeros((_BM, 1), jnp.int32)

    mm2 = lax.dot_general(
        zr_ref[...], cb_ref[...],
        dimension_numbers=(((1,), (1,)), ((), ())),
        preferred_element_type=jnp.float32,
    )
    dist = (zsq_ref[...] + csq_ref[...]) + mm2
    lmin = jnp.min(dist, axis=1, keepdims=True)
    lidx = jnp.min(jnp.where(dist == lmin, fiota_ref[...], jnp.float32(1e30)),
                   axis=1, keepdims=True).astype(jnp.int32)
    gidx = lidx + j * _BN
    improved = lmin < rmin[...]
    rmin[...] = jnp.where(improved, lmin, rmin[...])
    ridx[...] = jnp.where(improved, gidx, ridx[...])

    @pl.when(j == _N_CB - 1)
    def _():
        idx_ref[...] = ridx[...]


def _argmin_indices(z_r, z_sq, codebook, c_sq):
    fiota = jnp.arange(_BN, dtype=jnp.float32).reshape(1, _BN)
    return pl.pallas_call(
        _argmin_body,
        grid=(_N_TOK, _N_CB),
        in_specs=[
            pl.BlockSpec((_BM, _D), lambda i, j: (i, 0)),
            pl.BlockSpec((_BM, 1), lambda i, j: (i, 0)),
            pl.BlockSpec((_BN, _D), lambda i, j: (j, 0)),
            pl.BlockSpec((1, _BN), lambda i, j: (0, j)),
            pl.BlockSpec((1, _BN), lambda i, j: (0, 0)),
        ],
        out_specs=pl.BlockSpec((_BM, 1), lambda i, j: (i, 0)),
        out_shape=jax.ShapeDtypeStruct((_M, 1), jnp.int32),
        scratch_shapes=[
            pltpu.VMEM((_BM, 1), jnp.float32),
            pltpu.VMEM((_BM, 1), jnp.int32),
        ],
    )(z_r, z_sq, codebook, c_sq, fiota)


_NC = 2
_NS = 16
_NW = _NC * _NS
_B_PER_W = _M // _NW


def _sc_gather_body(table_hbm, idx_hbm, out_hbm, idx_v, rows_v, sem):
    wid = lax.axis_index("s") * _NC + lax.axis_index("c")
    base = wid * _B_PER_W
    pltpu.sync_copy(idx_hbm.at[pl.ds(base, _B_PER_W)], idx_v)
    pltpu.async_copy(table_hbm.at[idx_v], rows_v, sem).wait()
    pltpu.sync_copy(rows_v, out_hbm.at[pl.ds(base, _B_PER_W)])


def _sc_gather(codebook, idx):
    kern = functools.partial(
        pl.kernel,
        mesh=plsc.VectorSubcoreMesh(core_axis_name="c", subcore_axis_name="s"),
        out_type=jax.ShapeDtypeStruct((_M, _D), jnp.float32),
        scratch_types=[
            pltpu.VMEM((_B_PER_W,), jnp.int32),
            pltpu.VMEM((_B_PER_W, _D), jnp.float32),
            pltpu.SemaphoreType.DMA,
        ],
    )(_sc_gather_body)
    return kern(codebook, idx)


_N_ELEMS = _M * _D
_NB = 8
_SP = _M // _NB


def _ste_body(zr_ref, zq_ref, out_ref, loss_ref, acc):
    i = pl.program_id(0)

    @pl.when(i == 0)
    def _():
        acc[0, 0] = 0.0

    zmat = zr_ref[0]
    zqt = jnp.transpose(zq_ref[...], (1, 0))
    diff = zqt - zmat
    out_ref[0] = zmat + diff
    acc[0, 0] += jnp.sum(diff * diff)

    @pl.when(i == _NB - 1)
    def _():
        m = acc[0, 0] / _N_ELEMS
        loss_ref[...] = jnp.full((1, 1), _BETA * m + m, jnp.float32)


def _ste_and_loss(z_r, zq_flat):
    return pl.pallas_call(
        _ste_body,
        grid=(_NB,),
        in_specs=[
            pl.BlockSpec((1, _D, _SP), lambda i: (i, 0, 0)),
            pl.BlockSpec((_SP, _D), lambda i: (i, 0)),
        ],
        out_specs=[
            pl.BlockSpec((1, _D, _SP), lambda i: (i, 0, 0)),
            pl.BlockSpec((1, 1), lambda i: (0, 0)),
        ],
        out_shape=[
            jax.ShapeDtypeStruct((_NB, _D, _SP), jnp.float32),
            jax.ShapeDtypeStruct((1, 1), jnp.float32),
        ],
        scratch_shapes=[pltpu.SMEM((1, 1), jnp.float32)],
    )(z_r, zq_flat)


def kernel(z, codebook):
    z_sq = jnp.sum(jnp.moveaxis(z, 1, -1).reshape(-1, _D) ** 2,
                   axis=1, keepdims=True)
    c_sq = jnp.sum(codebook ** 2, axis=1).reshape(1, _K)
    z_r = z.reshape(_NB, _D, _SP)
    z2_flat = jnp.moveaxis(z, 1, -1).reshape(-1, _D) * (-2.0)

    idx2d = _argmin_indices(z2_flat, z_sq, codebook, c_sq)
    zq_flat = _sc_gather(codebook, idx2d.reshape(_M))
    out_r, loss = _ste_and_loss(z_r, zq_flat)

    z_q_out = out_r.reshape(z.shape)
    return (z_q_out, loss.reshape(()))


# speedup vs baseline: 1.3518x; 1.0799x over previous
import functools

import jax
import jax.numpy as jnp
from jax import lax
from jax.experimental import pallas as pl
from jax.experimental.pallas import tpu as pltpu
from jax.experimental.pallas import tpu_sc as plsc

_K = 8192
_D = 256
_BETA = 0.25
_M = 8192
_BM = 2048
_BN = 2048
_N_TOK = _M // _BM
_N_CB = _K // _BN


def _argmin_body(zr_ref, zsq_ref, cb_ref, csq_ref, fiota_ref, idx_ref,
                 rmin, ridx):
    j = pl.program_id(1)

    @pl.when(j == 0)
    def _():
        rmin[...] = jnp.full((_BM, 1), jnp.inf, jnp.float32)
        ridx[...] = jnp.zeros((_BM, 1), jnp.int32)

    mm2 = lax.dot_general(
        zr_ref[...], cb_ref[...],
        dimension_numbers=(((1,), (1,)), ((), ())),
        preferred_element_type=jnp.float32,
    )
    dist = (zsq_ref[...] + csq_ref[...]) + mm2
    lmin = jnp.min(dist, axis=1, keepdims=True)
    lidx = jnp.min(jnp.where(dist == lmin, fiota_ref[...], jnp.float32(1e30)),
                   axis=1, keepdims=True).astype(jnp.int32)
    gidx = lidx + j * _BN
    improved = lmin < rmin[...]
    rmin[...] = jnp.where(improved, lmin, rmin[...])
    ridx[...] = jnp.where(improved, gidx, ridx[...])

    @pl.when(j == _N_CB - 1)
    def _():
        idx_ref[...] = ridx[...]


def _argmin_indices(z_r, z_sq, codebook, c_sq):
    fiota = jnp.arange(_BN, dtype=jnp.float32).reshape(1, _BN)
    return pl.pallas_call(
        _argmin_body,
        grid=(_N_TOK, _N_CB),
        in_specs=[
            pl.BlockSpec((_BM, _D), lambda i, j: (i, 0)),
            pl.BlockSpec((_BM, 1), lambda i, j: (i, 0)),
            pl.BlockSpec((_BN, _D), lambda i, j: (j, 0)),
            pl.BlockSpec((1, _BN), lambda i, j: (0, j)),
            pl.BlockSpec((1, _BN), lambda i, j: (0, 0)),
        ],
        out_specs=pl.BlockSpec((_BM, 1), lambda i, j: (i, 0)),
        out_shape=jax.ShapeDtypeStruct((_M, 1), jnp.int32),
        scratch_shapes=[
            pltpu.VMEM((_BM, 1), jnp.float32),
            pltpu.VMEM((_BM, 1), jnp.int32),
        ],
    )(z_r, z_sq, codebook, c_sq, fiota)


_NC = 2
_NS = 16
_NW = _NC * _NS
_B_PER_W = _M // _NW


def _sc_gather_body(table_hbm, idx_hbm, out_hbm, idx_v, rows_v, sem):
    wid = lax.axis_index("s") * _NC + lax.axis_index("c")
    base = wid * _B_PER_W
    pltpu.sync_copy(idx_hbm.at[pl.ds(base, _B_PER_W)], idx_v)
    pltpu.async_copy(table_hbm.at[idx_v], rows_v, sem).wait()
    pltpu.sync_copy(rows_v, out_hbm.at[pl.ds(base, _B_PER_W)])


def _sc_gather(codebook, idx):
    kern = functools.partial(
        pl.kernel,
        mesh=plsc.VectorSubcoreMesh(core_axis_name="c", subcore_axis_name="s"),
        out_type=jax.ShapeDtypeStruct((_M, _D), jnp.float32),
        scratch_types=[
            pltpu.VMEM((_B_PER_W,), jnp.int32),
            pltpu.VMEM((_B_PER_W, _D), jnp.float32),
            pltpu.SemaphoreType.DMA,
        ],
    )(_sc_gather_body)
    return kern(codebook, idx)


_N_ELEMS = _M * _D
_NB = 8
_SP = _M // _NB


def _ste_body(z_ref, zq_ref, out_ref, loss_ref, acc):
    i = pl.program_id(0)

    @pl.when(i == 0)
    def _():
        acc[0, 0] = 0.0

    diff = zq_ref[...] - z_ref[...]
    out_ref[...] = z_ref[...] + diff
    acc[0, 0] += jnp.sum(diff * diff)

    @pl.when(i == _NB - 1)
    def _():
        m = acc[0, 0] / _N_ELEMS
        loss_ref[...] = jnp.full((1, 1), _BETA * m + m, jnp.float32)


def _ste_and_loss(z_flat, zq_flat):
    return pl.pallas_call(
        _ste_body,
        grid=(_NB,),
        in_specs=[
            pl.BlockSpec((_SP, _D), lambda i: (i, 0)),
            pl.BlockSpec((_SP, _D), lambda i: (i, 0)),
        ],
        out_specs=[
            pl.BlockSpec((_SP, _D), lambda i: (i, 0)),
            pl.BlockSpec((1, 1), lambda i: (0, 0)),
        ],
        out_shape=[
            jax.ShapeDtypeStruct((_M, _D), jnp.float32),
            jax.ShapeDtypeStruct((1, 1), jnp.float32),
        ],
        scratch_shapes=[pltpu.SMEM((1, 1), jnp.float32)],
    )(z_flat, zq_flat)


def kernel(z, codebook):
    z_flat = jnp.moveaxis(z, 1, -1).reshape(-1, _D)
    z_sq = jnp.sum(z_flat ** 2, axis=1, keepdims=True)
    c_sq = jnp.sum(codebook ** 2, axis=1).reshape(1, _K)

    idx2d = _argmin_indices(z_flat * (-2.0), z_sq, codebook, c_sq)
    zq_flat = _sc_gather(codebook, idx2d.reshape(_M))
    out_flat, loss = _ste_and_loss(z_flat, zq_flat)

    z_q_out = jnp.moveaxis(out_flat.reshape(_NB, 32, 32, _D), -1, 1)
    return (z_q_out, loss.reshape(()))


# speedup vs baseline: 1.3613x; 1.0070x over previous
import functools

import jax
import jax.numpy as jnp
from jax import lax
from jax.experimental import pallas as pl
from jax.experimental.pallas import tpu as pltpu
from jax.experimental.pallas import tpu_sc as plsc

_K = 8192
_D = 256
_BETA = 0.25
_M = 8192
_BM = 4096
_BN = 1024
_N_TOK = _M // _BM
_N_CB = _K // _BN


def _argmin_body(zr_ref, zsq_ref, cb_ref, csq_ref, fiota_ref, idx_ref,
                 rmin, ridx):
    j = pl.program_id(1)

    @pl.when(j == 0)
    def _():
        rmin[...] = jnp.full((_BM, 1), jnp.inf, jnp.float32)
        ridx[...] = jnp.zeros((_BM, 1), jnp.int32)

    mm2 = lax.dot_general(
        zr_ref[...], cb_ref[...],
        dimension_numbers=(((1,), (1,)), ((), ())),
        preferred_element_type=jnp.float32,
    )
    dist = (zsq_ref[...] + csq_ref[...]) + mm2
    lmin = jnp.min(dist, axis=1, keepdims=True)
    lidx = jnp.min(jnp.where(dist == lmin, fiota_ref[...], jnp.float32(1e30)),
                   axis=1, keepdims=True).astype(jnp.int32)
    gidx = lidx + j * _BN
    improved = lmin < rmin[...]
    rmin[...] = jnp.where(improved, lmin, rmin[...])
    ridx[...] = jnp.where(improved, gidx, ridx[...])

    @pl.when(j == _N_CB - 1)
    def _():
        idx_ref[...] = ridx[...]


def _argmin_indices(z_r, z_sq, codebook, c_sq):
    fiota = jnp.arange(_BN, dtype=jnp.float32).reshape(1, _BN)
    return pl.pallas_call(
        _argmin_body,
        grid=(_N_TOK, _N_CB),
        in_specs=[
            pl.BlockSpec((_BM, _D), lambda i, j: (i, 0)),
            pl.BlockSpec((_BM, 1), lambda i, j: (i, 0)),
            pl.BlockSpec((_BN, _D), lambda i, j: (j, 0)),
            pl.BlockSpec((1, _BN), lambda i, j: (0, j)),
            pl.BlockSpec((1, _BN), lambda i, j: (0, 0)),
        ],
        out_specs=pl.BlockSpec((_BM, 1), lambda i, j: (i, 0)),
        out_shape=jax.ShapeDtypeStruct((_M, 1), jnp.int32),
        scratch_shapes=[
            pltpu.VMEM((_BM, 1), jnp.float32),
            pltpu.VMEM((_BM, 1), jnp.int32),
        ],
    )(z_r, z_sq, codebook, c_sq, fiota)


_NC = 2
_NS = 16
_NW = _NC * _NS
_B_PER_W = _M // _NW


def _sc_gather_body(table_hbm, idx_hbm, out_hbm, idx_v, rows_v, sem):
    wid = lax.axis_index("s") * _NC + lax.axis_index("c")
    base = wid * _B_PER_W
    pltpu.sync_copy(idx_hbm.at[pl.ds(base, _B_PER_W)], idx_v)
    pltpu.async_copy(table_hbm.at[idx_v], rows_v, sem).wait()
    pltpu.sync_copy(rows_v, out_hbm.at[pl.ds(base, _B_PER_W)])


def _sc_gather(codebook, idx):
    kern = functools.partial(
        pl.kernel,
        mesh=plsc.VectorSubcoreMesh(core_axis_name="c", subcore_axis_name="s"),
        out_type=jax.ShapeDtypeStruct((_M, _D), jnp.float32),
        scratch_types=[
            pltpu.VMEM((_B_PER_W,), jnp.int32),
            pltpu.VMEM((_B_PER_W, _D), jnp.float32),
            pltpu.SemaphoreType.DMA,
        ],
    )(_sc_gather_body)
    return kern(codebook, idx)


_N_ELEMS = _M * _D
_NB = 8
_SP = _M // _NB


def _ste_body(z_ref, zq_ref, out_ref, loss_ref, acc):
    i = pl.program_id(0)

    @pl.when(i == 0)
    def _():
        acc[0, 0] = 0.0

    diff = zq_ref[...] - z_ref[...]
    out_ref[...] = z_ref[...] + diff
    acc[0, 0] += jnp.sum(diff * diff)

    @pl.when(i == _NB - 1)
    def _():
        m = acc[0, 0] / _N_ELEMS
        loss_ref[...] = jnp.full((1, 1), _BETA * m + m, jnp.float32)


def _ste_and_loss(z_flat, zq_flat):
    return pl.pallas_call(
        _ste_body,
        grid=(_NB,),
        in_specs=[
            pl.BlockSpec((_SP, _D), lambda i: (i, 0)),
            pl.BlockSpec((_SP, _D), lambda i: (i, 0)),
        ],
        out_specs=[
            pl.BlockSpec((_SP, _D), lambda i: (i, 0)),
            pl.BlockSpec((1, 1), lambda i: (0, 0)),
        ],
        out_shape=[
            jax.ShapeDtypeStruct((_M, _D), jnp.float32),
            jax.ShapeDtypeStruct((1, 1), jnp.float32),
        ],
        scratch_shapes=[pltpu.SMEM((1, 1), jnp.float32)],
    )(z_flat, zq_flat)


def kernel(z, codebook):
    z_flat = jnp.moveaxis(z, 1, -1).reshape(-1, _D)
    z_sq = jnp.sum(z_flat ** 2, axis=1, keepdims=True)
    c_sq = jnp.sum(codebook ** 2, axis=1).reshape(1, _K)

    idx2d = _argmin_indices(z_flat * (-2.0), z_sq, codebook, c_sq)
    zq_flat = _sc_gather(codebook, idx2d.reshape(_M))
    out_flat, loss = _ste_and_loss(z_flat, zq_flat)

    z_q_out = jnp.moveaxis(out_flat.reshape(_NB, 32, 32, _D), -1, 1)
    return (z_q_out, loss.reshape(()))
